# baseline (speedup 1.0000x reference)
import functools

import jax
import jax.numpy as jnp
from jax import lax
from jax.experimental import pallas as pl
from jax.experimental.pallas import tpu as pltpu

_N_ROUNDS = 20
_K = 20
_SCORE_THRESHOLD = 0.3
_BIG_M = 4096
_BIG_P = 8192


def _nms_body(bx_ref, by_ref, bw_ref, bh_ref, prob_ref, noise_ref, scal_ref,
              chosen_ref, idx_ref, k0_ref, maskbf_ref, *, n_real, n_pad):
    f32 = jnp.float32
    i32 = jnp.int32
    bx = bx_ref[0]
    by = by_ref[0]
    bw = bw_ref[0]
    bh = bh_ref[0]
    prob = prob_ref[0]
    noise = noise_ref[0]
    thr = scal_ref[0, 0, 0]
    factor = scal_ref[0, 0, 1]
    topk_only = scal_ref[0, 0, 2]

    score = jnp.maximum(prob + factor * noise, 0.0)

    x1 = bx - 0.5 * bw
    x3 = bx + 0.5 * bw
    y1 = by - 0.5 * bh
    y3 = by + 0.5 * bh
    area = bw * bh

    def to_col(row):
        return jnp.transpose(row, (1, 0))

    iota_col = lax.broadcasted_iota(i32, (n_pad, 1), 0)
    iota_row = lax.broadcasted_iota(i32, (1, n_pad), 1)

    score_col = to_col(score)
    tie = (score_col == score) & (iota_col < iota_row)
    rank_mat = (score_col > score).astype(i32) + tie.astype(i32)
    rank0 = jnp.sum(rank_mat, axis=0, keepdims=True)
    rank0_col = to_col(rank0)

    x1c, x3c = to_col(x1), to_col(x3)
    y1c, y3c = to_col(y1), to_col(y3)
    areac = to_col(area)
    xi1 = jnp.maximum(x1, x1c)
    yi1 = jnp.maximum(y1, y1c)
    xi3 = jnp.minimum(x3, x3c)
    yi3 = jnp.minimum(y3, y3c)
    inter = jnp.maximum(xi3 - xi1, 0.0) * jnp.maximum(yi3 - yi1, 0.0)
    min_area = jnp.minimum(area, areac)
    maskb = (inter / min_area) > thr
    k0_ref[...] = jnp.where(maskb, jnp.broadcast_to(rank0, (n_pad, n_pad)),
                            _BIG_M)
    maskbf_ref[...] = maskb.astype(jnp.bfloat16)

    possible0 = jnp.where(score > _SCORE_THRESHOLD, 1.0, 0.0)
    selected0 = jnp.zeros((n_pad, 1), f32)

    def round_body(_, carry):
        possible_row, selected_col = carry

        def active(args):
            possible_row, selected_col = args
            pen = jnp.where(possible_row > 0.0, 0, _BIG_P).astype(i32)
            am = jnp.min(k0_ref[...] + pen, axis=1, keepdims=True)
            possible_col = to_col(possible_row)
            no_nbr = am >= _BIG_M
            newly_cond = (am == rank0_col) | (no_nbr & (iota_col == 0))
            newly = jnp.where(newly_cond, possible_col, 0.0)
            selected2 = selected_col + newly
            blocked = jnp.dot(maskbf_ref[...], newly.astype(jnp.bfloat16),
                              preferred_element_type=f32)
            blocked_row = jnp.transpose(blocked, (1, 0))
            possible2 = jnp.where(blocked_row == 0.0, possible_row, 0.0)
            return possible2, selected2

        any_possible = jnp.sum(possible_row) > 0.0
        return lax.cond(any_possible, active, lambda a: a,
                        (possible_row, selected_col))

    possible_row, selected_col = lax.fori_loop(
        0, _N_ROUNDS, round_body, (possible0, selected0))

    selected_row = jnp.transpose(selected_col, (1, 0))
    chosen = jnp.where(topk_only != 0.0, 1.0, selected_row)
    chosen_ref[0] = chosen

    masked = jnp.where(iota_row < n_real, chosen * score, -1.0)
    idx_vec = jnp.zeros((1, 128), jnp.int32)
    lane128 = lax.broadcasted_iota(jnp.int32, (1, 128), 1)
    for k in range(_K):
        m = jnp.max(masked)
        am = jnp.min(jnp.where(masked == m, iota_row, n_pad))
        idx_vec = jnp.where(lane128 == k, am, idx_vec)
        masked = jnp.where(iota_row == am, -1.0, masked)
    idx_ref[0] = idx_vec


def kernel(prob, bx, by, bw, bh, overlap_threshold, randomize_nms_factor,
           n_objects_max, topk_only):
    n, b = prob.shape[0], prob.shape[1]
    n_pad = ((n + 127) // 128) * 128

    def prep(a, pad_val):
        a2 = jnp.transpose(a[..., 0], (1, 0))
        return jnp.pad(a2, ((0, 0), (0, n_pad - n)),
                       constant_values=pad_val).reshape(b, 1, n_pad)

    bx_p = prep(bx, -100.0)
    by_p = prep(by, -100.0)
    bw_p = prep(bw, 0.0)
    bh_p = prep(bh, 0.0)
    prob_p = prep(prob, 0.0)

    noise = jax.random.normal(jax.random.key(42), (n, b), dtype=jnp.float32)
    noise_p = jnp.pad(noise.T, ((0, 0), (0, n_pad - n))).reshape(b, 1, n_pad)

    scal = jnp.zeros((1, 1, 128), jnp.float32)
    scal = scal.at[0, 0, 0].set(overlap_threshold[0])
    scal = scal.at[0, 0, 1].set(randomize_nms_factor[0])
    scal = scal.at[0, 0, 2].set(jnp.asarray(topk_only).astype(jnp.float32))

    body = functools.partial(_nms_body, n_real=n, n_pad=n_pad)
    chosen_b, idx_b = pl.pallas_call(
        body,
        grid=(b,),
        in_specs=[
            pl.BlockSpec((1, 1, n_pad), lambda i: (i, 0, 0)),
            pl.BlockSpec((1, 1, n_pad), lambda i: (i, 0, 0)),
            pl.BlockSpec((1, 1, n_pad), lambda i: (i, 0, 0)),
            pl.BlockSpec((1, 1, n_pad), lambda i: (i, 0, 0)),
            pl.BlockSpec((1, 1, n_pad), lambda i: (i, 0, 0)),
            pl.BlockSpec((1, 1, n_pad), lambda i: (i, 0, 0)),
            pl.BlockSpec((1, 1, 128), lambda i: (0, 0, 0)),
        ],
        out_specs=[
            pl.BlockSpec((1, 1, n_pad), lambda i: (i, 0, 0)),
            pl.BlockSpec((1, 1, 128), lambda i: (i, 0, 0)),
        ],
        out_shape=[
            jax.ShapeDtypeStruct((b, 1, n_pad), jnp.float32),
            jax.ShapeDtypeStruct((b, 1, 128), jnp.int32),
        ],
        scratch_shapes=[
            pltpu.VMEM((n_pad, n_pad), jnp.int32),
            pltpu.VMEM((n_pad, n_pad), jnp.bfloat16),
        ],
        compiler_params=pltpu.CompilerParams(
            dimension_semantics=("arbitrary",)),
    )(bx_p, by_p, bw_p, bh_p, prob_p, noise_p, scal)

    chosen = chosen_b.reshape(b, n_pad)[:, :n].T
    top_k_indices = idx_b.reshape(b, 128)[:, :_K].T
    batch_indices = jnp.broadcast_to(
        jnp.arange(b, dtype=top_k_indices.dtype).reshape(1, -1), (_K, b))
    return chosen, top_k_indices, batch_indices


# speedup vs baseline: 1.6279x; 1.6279x over previous
import functools

import jax
import jax.numpy as jnp
from jax import lax
from jax.experimental import pallas as pl
from jax.experimental.pallas import tpu as pltpu

_N_ROUNDS = 20
_K = 20
_SCORE_THRESHOLD = 0.3
_BIG_M = 4096
_BIG_P = 8192
_PAIR = 2


def _nms_body(bx_ref, by_ref, bw_ref, bh_ref, prob_ref, noise_ref, scal_ref,
              chosen_ref, idx_ref, k0_ref, maskbf_ref, *, n_real, n_pad):
    f32 = jnp.float32
    i32 = jnp.int32
    thr = scal_ref[0, 0, 0]
    factor = scal_ref[0, 0, 1]
    topk_only = scal_ref[0, 0, 2]

    def to_col(row):
        return jnp.transpose(row, (1, 0))

    iota_col = lax.broadcasted_iota(i32, (n_pad, 1), 0)
    iota_row = lax.broadcasted_iota(i32, (1, n_pad), 1)

    scores = []
    rank0_cols = []
    for t in range(_PAIR):
        bx = bx_ref[t]
        by = by_ref[t]
        bw = bw_ref[t]
        bh = bh_ref[t]
        prob = prob_ref[t]
        noise = noise_ref[t]

        score = jnp.maximum(prob + factor * noise, 0.0)
        scores.append(score)

        x1 = bx - 0.5 * bw
        x3 = bx + 0.5 * bw
        y1 = by - 0.5 * bh
        y3 = by + 0.5 * bh
        area = bw * bh

        score_col = to_col(score)
        tie = (score_col == score) & (iota_col < iota_row)
        rank_mat = (score_col > score).astype(i32) + tie.astype(i32)
        rank0 = jnp.sum(rank_mat, axis=0, keepdims=True)
        rank0_cols.append(to_col(rank0))

        x1c, x3c = to_col(x1), to_col(x3)
        y1c, y3c = to_col(y1), to_col(y3)
        areac = to_col(area)
        xi1 = jnp.maximum(x1, x1c)
        yi1 = jnp.maximum(y1, y1c)
        xi3 = jnp.minimum(x3, x3c)
        yi3 = jnp.minimum(y3, y3c)
        inter = jnp.maximum(xi3 - xi1, 0.0) * jnp.maximum(yi3 - yi1, 0.0)
        min_area = jnp.minimum(area, areac)
        maskb = (inter / min_area) > thr
        k0_ref[t] = jnp.where(
            maskb, jnp.broadcast_to(rank0, (n_pad, n_pad)), _BIG_M)
        maskbf_ref[t] = maskb.astype(jnp.bfloat16)

    possibles0 = tuple(
        jnp.where(scores[t] > _SCORE_THRESHOLD, 1.0, 0.0) for t in range(_PAIR))
    selecteds0 = tuple(jnp.zeros((n_pad, 1), f32) for _ in range(_PAIR))

    def cond_fun(carry):
        t, possibles, _ = carry
        alive = sum(jnp.sum(p) for p in possibles)
        return (t < _N_ROUNDS) & (alive > 0.0)

    def body_fun(carry):
        t, possibles, selecteds = carry
        new_p, new_s = [], []
        for u in range(_PAIR):
            possible_row = possibles[u]
            selected_col = selecteds[u]
            pen = jnp.where(possible_row > 0.0, 0, _BIG_P).astype(i32)
            am = jnp.min(k0_ref[u] + pen, axis=1, keepdims=True)
            possible_col = to_col(possible_row)
            no_nbr = am >= _BIG_M
            newly_cond = (am == rank0_cols[u]) | (no_nbr & (iota_col == 0))
            newly = jnp.where(newly_cond, possible_col, 0.0)
            blocked = jnp.dot(maskbf_ref[u], newly.astype(jnp.bfloat16),
                              preferred_element_type=f32)
            blocked_row = jnp.transpose(blocked, (1, 0))
            new_p.append(jnp.where(blocked_row == 0.0, possible_row, 0.0))
            new_s.append(selected_col + newly)
        return t + 1, tuple(new_p), tuple(new_s)

    _, possibles, selecteds = lax.while_loop(
        cond_fun, body_fun, (jnp.int32(0), possibles0, selecteds0))

    score2 = jnp.concatenate(scores, axis=0)
    selected2 = jnp.concatenate(
        [jnp.transpose(s, (1, 0)) for s in selecteds], axis=0)
    chosen = jnp.where(topk_only != 0.0, 1.0, selected2)
    chosen_ref[...] = chosen.reshape(_PAIR, 1, n_pad)

    masked = jnp.where(iota_row < n_real, chosen * score2, -1.0)
    idx_vec = jnp.zeros((_PAIR, 128), jnp.int32)
    lane128 = lax.broadcasted_iota(jnp.int32, (_PAIR, 128), 1)
    iota_row2 = jnp.broadcast_to(iota_row, (_PAIR, n_pad))
    for k in range(_K):
        m = jnp.max(masked, axis=1, keepdims=True)
        am = jnp.min(jnp.where(masked == m, iota_row2, n_pad),
                     axis=1, keepdims=True)
        idx_vec = jnp.where(lane128 == k, am, idx_vec)
        masked = jnp.where(iota_row2 == am, -1.0, masked)
    idx_ref[...] = idx_vec.reshape(_PAIR, 1, 128)


def kernel(prob, bx, by, bw, bh, overlap_threshold, randomize_nms_factor,
           n_objects_max, topk_only):
    n, b = prob.shape[0], prob.shape[1]
    n_pad = ((n + 127) // 128) * 128

    def prep(a, pad_val):
        a2 = jnp.transpose(a[..., 0], (1, 0))
        return jnp.pad(a2, ((0, 0), (0, n_pad - n)),
                       constant_values=pad_val).reshape(b, 1, n_pad)

    bx_p = prep(bx, -100.0)
    by_p = prep(by, -100.0)
    bw_p = prep(bw, 0.0)
    bh_p = prep(bh, 0.0)
    prob_p = prep(prob, 0.0)

    noise = jax.random.normal(jax.random.key(42), (n, b), dtype=jnp.float32)
    noise_p = jnp.pad(noise.T, ((0, 0), (0, n_pad - n))).reshape(b, 1, n_pad)

    scal = jnp.zeros((1, 1, 128), jnp.float32)
    scal = scal.at[0, 0, 0].set(overlap_threshold[0])
    scal = scal.at[0, 0, 1].set(randomize_nms_factor[0])
    scal = scal.at[0, 0, 2].set(jnp.asarray(topk_only).astype(jnp.float32))

    body = functools.partial(_nms_body, n_real=n, n_pad=n_pad)
    grid = b // _PAIR
    chosen_b, idx_b = pl.pallas_call(
        body,
        grid=(grid,),
        in_specs=[
            pl.BlockSpec((_PAIR, 1, n_pad), lambda i: (i, 0, 0)),
            pl.BlockSpec((_PAIR, 1, n_pad), lambda i: (i, 0, 0)),
            pl.BlockSpec((_PAIR, 1, n_pad), lambda i: (i, 0, 0)),
            pl.BlockSpec((_PAIR, 1, n_pad), lambda i: (i, 0, 0)),
            pl.BlockSpec((_PAIR, 1, n_pad), lambda i: (i, 0, 0)),
            pl.BlockSpec((_PAIR, 1, n_pad), lambda i: (i, 0, 0)),
            pl.BlockSpec((1, 1, 128), lambda i: (0, 0, 0)),
        ],
        out_specs=[
            pl.BlockSpec((_PAIR, 1, n_pad), lambda i: (i, 0, 0)),
            pl.BlockSpec((_PAIR, 1, 128), lambda i: (i, 0, 0)),
        ],
        out_shape=[
            jax.ShapeDtypeStruct((b, 1, n_pad), jnp.float32),
            jax.ShapeDtypeStruct((b, 1, 128), jnp.int32),
        ],
        scratch_shapes=[
            pltpu.VMEM((_PAIR, n_pad, n_pad), jnp.int32),
            pltpu.VMEM((_PAIR, n_pad, n_pad), jnp.bfloat16),
        ],
        compiler_params=pltpu.CompilerParams(
            dimension_semantics=("parallel",)),
    )(bx_p, by_p, bw_p, bh_p, prob_p, noise_p, scal)

    chosen = chosen_b.reshape(b, n_pad)[:, :n].T
    top_k_indices = idx_b.reshape(b, 128)[:, :_K].T
    batch_indices = jnp.broadcast_to(
        jnp.arange(b, dtype=top_k_indices.dtype).reshape(1, -1), (_K, b))
    return chosen, top_k_indices, batch_indices


# speedup vs baseline: 1.7946x; 1.1024x over previous
import functools

import jax
import jax.numpy as jnp
from jax import lax
from jax.experimental import pallas as pl
from jax.experimental.pallas import tpu as pltpu

_N_ROUNDS = 20
_K = 20
_SCORE_THRESHOLD = 0.3
_BIG_M = 4096
_BIG_P = 8192
_PAIR = 4


def _nms_body(bx_ref, by_ref, bw_ref, bh_ref, prob_ref, noise_ref, scal_ref,
              chosen_ref, idx_ref, k0_ref, maskbf_ref, *, n_real, n_pad):
    f32 = jnp.float32
    i32 = jnp.int32
    thr = scal_ref[0, 0, 0]
    factor = scal_ref[0, 0, 1]
    topk_only = scal_ref[0, 0, 2]

    def to_col(row):
        return jnp.transpose(row, (1, 0))

    iota_col = lax.broadcasted_iota(i32, (n_pad, 1), 0)
    iota_row = lax.broadcasted_iota(i32, (1, n_pad), 1)

    scores = []
    rank0_cols = []
    for t in range(_PAIR):
        bx = bx_ref[t]
        by = by_ref[t]
        bw = bw_ref[t]
        bh = bh_ref[t]
        prob = prob_ref[t]
        noise = noise_ref[t]

        score = jnp.maximum(prob + factor * noise, 0.0)
        scores.append(score)

        x1 = bx - 0.5 * bw
        x3 = bx + 0.5 * bw
        y1 = by - 0.5 * bh
        y3 = by + 0.5 * bh
        area = bw * bh

        score_col = to_col(score)
        tie = (score_col == score) & (iota_col < iota_row)
        rank_mat = (score_col > score).astype(i32) + tie.astype(i32)
        rank0 = jnp.sum(rank_mat, axis=0, keepdims=True)
        rank0_cols.append(to_col(rank0))

        x1c, x3c = to_col(x1), to_col(x3)
        y1c, y3c = to_col(y1), to_col(y3)
        areac = to_col(area)
        xi1 = jnp.maximum(x1, x1c)
        yi1 = jnp.maximum(y1, y1c)
        xi3 = jnp.minimum(x3, x3c)
        yi3 = jnp.minimum(y3, y3c)
        inter = jnp.maximum(xi3 - xi1, 0.0) * jnp.maximum(yi3 - yi1, 0.0)
        min_area = jnp.minimum(area, areac)
        maskb = (inter / min_area) > thr
        k0_ref[t] = jnp.where(
            maskb, jnp.broadcast_to(rank0, (n_pad, n_pad)), _BIG_M)
        maskbf_ref[t] = maskb.astype(jnp.bfloat16)

    possibles0 = tuple(
        jnp.where(scores[t] > _SCORE_THRESHOLD, 1.0, 0.0) for t in range(_PAIR))
    selecteds0 = tuple(jnp.zeros((n_pad, 1), f32) for _ in range(_PAIR))

    def cond_fun(carry):
        t, possibles, _ = carry
        alive = sum(jnp.sum(p) for p in possibles)
        return (t < _N_ROUNDS) & (alive > 0.0)

    def body_fun(carry):
        t, possibles, selecteds = carry
        new_p, new_s = [], []
        for u in range(_PAIR):
            possible_row = possibles[u]
            selected_col = selecteds[u]
            pen = jnp.where(possible_row > 0.0, 0, _BIG_P).astype(i32)
            am = jnp.min(k0_ref[u] + pen, axis=1, keepdims=True)
            possible_col = to_col(possible_row)
            no_nbr = am >= _BIG_M
            newly_cond = (am == rank0_cols[u]) | (no_nbr & (iota_col == 0))
            newly = jnp.where(newly_cond, possible_col, 0.0)
            blocked = jnp.dot(maskbf_ref[u], newly.astype(jnp.bfloat16),
                              preferred_element_type=f32)
            blocked_row = jnp.transpose(blocked, (1, 0))
            new_p.append(jnp.where(blocked_row == 0.0, possible_row, 0.0))
            new_s.append(selected_col + newly)
        return t + 1, tuple(new_p), tuple(new_s)

    _, possibles, selecteds = lax.while_loop(
        cond_fun, body_fun, (jnp.int32(0), possibles0, selecteds0))

    score2 = jnp.concatenate(scores, axis=0)
    selected2 = jnp.concatenate(
        [jnp.transpose(s, (1, 0)) for s in selecteds], axis=0)
    chosen = jnp.where(topk_only != 0.0, 1.0, selected2)
    chosen_ref[...] = chosen.reshape(_PAIR, 1, n_pad)

    masked = jnp.where(iota_row < n_real, chosen * score2, -1.0)
    idx_vec = jnp.zeros((_PAIR, 128), jnp.int32)
    lane128 = lax.broadcasted_iota(jnp.int32, (_PAIR, 128), 1)
    iota_row2 = jnp.broadcast_to(iota_row, (_PAIR, n_pad))
    for k in range(_K):
        m = jnp.max(masked, axis=1, keepdims=True)
        am = jnp.min(jnp.where(masked == m, iota_row2, n_pad),
                     axis=1, keepdims=True)
        idx_vec = jnp.where(lane128 == k, am, idx_vec)
        masked = jnp.where(iota_row2 == am, -1.0, masked)
    idx_ref[...] = idx_vec.reshape(_PAIR, 1, 128)


def kernel(prob, bx, by, bw, bh, overlap_threshold, randomize_nms_factor,
           n_objects_max, topk_only):
    n, b = prob.shape[0], prob.shape[1]
    n_pad = ((n + 127) // 128) * 128

    def prep(a, pad_val):
        a2 = jnp.transpose(a[..., 0], (1, 0))
        return jnp.pad(a2, ((0, 0), (0, n_pad - n)),
                       constant_values=pad_val).reshape(b, 1, n_pad)

    bx_p = prep(bx, -100.0)
    by_p = prep(by, -100.0)
    bw_p = prep(bw, 0.0)
    bh_p = prep(bh, 0.0)
    prob_p = prep(prob, 0.0)

    noise = jax.random.normal(jax.random.key(42), (n, b), dtype=jnp.float32)
    noise_p = jnp.pad(noise.T, ((0, 0), (0, n_pad - n))).reshape(b, 1, n_pad)

    scal = jnp.zeros((1, 1, 128), jnp.float32)
    scal = scal.at[0, 0, 0].set(overlap_threshold[0])
    scal = scal.at[0, 0, 1].set(randomize_nms_factor[0])
    scal = scal.at[0, 0, 2].set(jnp.asarray(topk_only).astype(jnp.float32))

    body = functools.partial(_nms_body, n_real=n, n_pad=n_pad)
    grid = b // _PAIR
    chosen_b, idx_b = pl.pallas_call(
        body,
        grid=(grid,),
        in_specs=[
            pl.BlockSpec((_PAIR, 1, n_pad), lambda i: (i, 0, 0)),
            pl.BlockSpec((_PAIR, 1, n_pad), lambda i: (i, 0, 0)),
            pl.BlockSpec((_PAIR, 1, n_pad), lambda i: (i, 0, 0)),
            pl.BlockSpec((_PAIR, 1, n_pad), lambda i: (i, 0, 0)),
            pl.BlockSpec((_PAIR, 1, n_pad), lambda i: (i, 0, 0)),
            pl.BlockSpec((_PAIR, 1, n_pad), lambda i: (i, 0, 0)),
            pl.BlockSpec((1, 1, 128), lambda i: (0, 0, 0)),
        ],
        out_specs=[
            pl.BlockSpec((_PAIR, 1, n_pad), lambda i: (i, 0, 0)),
            pl.BlockSpec((_PAIR, 1, 128), lambda i: (i, 0, 0)),
        ],
        out_shape=[
            jax.ShapeDtypeStruct((b, 1, n_pad), jnp.float32),
            jax.ShapeDtypeStruct((b, 1, 128), jnp.int32),
        ],
        scratch_shapes=[
            pltpu.VMEM((_PAIR, n_pad, n_pad), jnp.int32),
            pltpu.VMEM((_PAIR, n_pad, n_pad), jnp.bfloat16),
        ],
        compiler_params=pltpu.CompilerParams(
            dimension_semantics=("parallel",)),
    )(bx_p, by_p, bw_p, bh_p, prob_p, noise_p, scal)

    chosen = chosen_b.reshape(b, n_pad)[:, :n].T
    top_k_indices = idx_b.reshape(b, 128)[:, :_K].T
    batch_indices = jnp.broadcast_to(
        jnp.arange(b, dtype=top_k_indices.dtype).reshape(1, -1), (_K, b))
    return chosen, top_k_indices, batch_indices


# speedup vs baseline: 1.8114x; 1.0094x over previous
import functools

import jax
import jax.numpy as jnp
from jax import lax
from jax.experimental import pallas as pl
from jax.experimental.pallas import tpu as pltpu

_N_ROUNDS = 20
_K = 20
_SCORE_THRESHOLD = 0.3
_BIG_M = 4096
_BIG_P = 8192
_PAIR = 4
_STRIP = 32


def _nms_body(bx_ref, by_ref, bw_ref, bh_ref, prob_ref, noise_ref, scal_ref,
              chosen_ref, idx_ref, k0_ref, maskbf_ref, *, n_real, n_pad):
    f32 = jnp.float32
    i32 = jnp.int32
    thr = scal_ref[0, 0, 0]
    factor = scal_ref[0, 0, 1]
    topk_only = scal_ref[0, 0, 2]
    n_strips = n_pad // _STRIP

    def to_col(row):
        return jnp.transpose(row, (1, 0))

    iota_col = lax.broadcasted_iota(i32, (n_pad, 1), 0)
    iota_row = lax.broadcasted_iota(i32, (1, n_pad), 1)

    scores = []
    rank0_cols = []
    for t in range(_PAIR):
        bx = bx_ref[t]
        by = by_ref[t]
        bw = bw_ref[t]
        bh = bh_ref[t]
        prob = prob_ref[t]
        noise = noise_ref[t]

        score = jnp.maximum(prob + factor * noise, 0.0)
        scores.append(score)

        x1 = bx - 0.5 * bw
        x3 = bx + 0.5 * bw
        y1 = by - 0.5 * bh
        y3 = by + 0.5 * bh
        area = bw * bh

        score_col = to_col(score)
        x1c, x3c = to_col(x1), to_col(x3)
        y1c, y3c = to_col(y1), to_col(y3)
        areac = to_col(area)

        rank0 = jnp.zeros((1, n_pad), i32)
        for s in range(n_strips):
            sl = slice(s * _STRIP, (s + 1) * _STRIP)
            sc_s = score_col[sl]
            io_s = iota_col[sl]
            tie = (sc_s == score) & (io_s < iota_row)
            part = (sc_s > score).astype(i32) + tie.astype(i32)
            rank0 = rank0 + jnp.sum(part, axis=0, keepdims=True)
        rank0_cols.append(to_col(rank0))
        rank0_b = jnp.broadcast_to(rank0, (_STRIP, n_pad))

        for s in range(n_strips):
            sl = slice(s * _STRIP, (s + 1) * _STRIP)
            xi1 = jnp.maximum(x1, x1c[sl])
            yi1 = jnp.maximum(y1, y1c[sl])
            xi3 = jnp.minimum(x3, x3c[sl])
            yi3 = jnp.minimum(y3, y3c[sl])
            inter = (jnp.maximum(xi3 - xi1, 0.0)
                     * jnp.maximum(yi3 - yi1, 0.0))
            min_area = jnp.minimum(area, areac[sl])
            maskb = (inter / min_area) > thr
            k0_ref[t, sl, :] = jnp.where(maskb, rank0_b, _BIG_M)
            maskbf_ref[t, sl, :] = maskb.astype(jnp.bfloat16)

    possibles0 = tuple(
        jnp.where(scores[t] > _SCORE_THRESHOLD, 1.0, 0.0) for t in range(_PAIR))
    selecteds0 = tuple(jnp.zeros((n_pad, 1), f32) for _ in range(_PAIR))

    def cond_fun(carry):
        t, possibles, _ = carry
        alive = sum(jnp.sum(p) for p in possibles)
        return (t < _N_ROUNDS) & (alive > 0.0)

    def body_fun(carry):
        t, possibles, selecteds = carry
        new_p, new_s = [], []
        for u in range(_PAIR):
            possible_row = possibles[u]
            selected_col = selecteds[u]
            pen = jnp.where(possible_row > 0.0, 0, _BIG_P).astype(i32)
            parts = []
            for s in range(n_strips):
                sl = slice(s * _STRIP, (s + 1) * _STRIP)
                key_s = k0_ref[u, sl, :] + pen
                parts.append(jnp.min(key_s, axis=1, keepdims=True))
            am = jnp.concatenate(parts, axis=0)
            possible_col = to_col(possible_row)
            no_nbr = am >= _BIG_M
            newly_cond = (am == rank0_cols[u]) | (no_nbr & (iota_col == 0))
            newly = jnp.where(newly_cond, possible_col, 0.0)
            blocked = jnp.dot(maskbf_ref[u], newly.astype(jnp.bfloat16),
                              preferred_element_type=f32)
            blocked_row = jnp.transpose(blocked, (1, 0))
            new_p.append(jnp.where(blocked_row == 0.0, possible_row, 0.0))
            new_s.append(selected_col + newly)
        return t + 1, tuple(new_p), tuple(new_s)

    _, possibles, selecteds = lax.while_loop(
        cond_fun, body_fun, (jnp.int32(0), possibles0, selecteds0))

    score2 = jnp.concatenate(scores, axis=0)
    selected2 = jnp.concatenate(
        [jnp.transpose(s, (1, 0)) for s in selecteds], axis=0)
    chosen = jnp.where(topk_only != 0.0, 1.0, selected2)
    chosen_ref[...] = chosen.reshape(_PAIR, 1, n_pad)

    masked = jnp.where(iota_row < n_real, chosen * score2, -1.0)
    idx_vec = jnp.zeros((_PAIR, 128), jnp.int32)
    lane128 = lax.broadcasted_iota(jnp.int32, (_PAIR, 128), 1)
    iota_row2 = jnp.broadcast_to(iota_row, (_PAIR, n_pad))
    for k in range(_K):
        m = jnp.max(masked, axis=1, keepdims=True)
        am = jnp.min(jnp.where(masked == m, iota_row2, n_pad),
                     axis=1, keepdims=True)
        idx_vec = jnp.where(lane128 == k, am, idx_vec)
        masked = jnp.where(iota_row2 == am, -1.0, masked)
    idx_ref[...] = idx_vec.reshape(_PAIR, 1, 128)


def kernel(prob, bx, by, bw, bh, overlap_threshold, randomize_nms_factor,
           n_objects_max, topk_only):
    n, b = prob.shape[0], prob.shape[1]
    n_pad = ((n + 127) // 128) * 128

    def prep(a, pad_val):
        a2 = jnp.transpose(a[..., 0], (1, 0))
        return jnp.pad(a2, ((0, 0), (0, n_pad - n)),
                       constant_values=pad_val).reshape(b, 1, n_pad)

    bx_p = prep(bx, -100.0)
    by_p = prep(by, -100.0)
    bw_p = prep(bw, 0.0)
    bh_p = prep(bh, 0.0)
    prob_p = prep(prob, 0.0)

    noise = jax.random.normal(jax.random.key(42), (n, b), dtype=jnp.float32)
    noise_p = jnp.pad(noise.T, ((0, 0), (0, n_pad - n))).reshape(b, 1, n_pad)

    scal = jnp.zeros((1, 1, 128), jnp.float32)
    scal = scal.at[0, 0, 0].set(overlap_threshold[0])
    scal = scal.at[0, 0, 1].set(randomize_nms_factor[0])
    scal = scal.at[0, 0, 2].set(jnp.asarray(topk_only).astype(jnp.float32))

    body = functools.partial(_nms_body, n_real=n, n_pad=n_pad)
    grid = b // _PAIR
    chosen_b, idx_b = pl.pallas_call(
        body,
        grid=(grid,),
        in_specs=[
            pl.BlockSpec((_PAIR, 1, n_pad), lambda i: (i, 0, 0)),
            pl.BlockSpec((_PAIR, 1, n_pad), lambda i: (i, 0, 0)),
            pl.BlockSpec((_PAIR, 1, n_pad), lambda i: (i, 0, 0)),
            pl.BlockSpec((_PAIR, 1, n_pad), lambda i: (i, 0, 0)),
            pl.BlockSpec((_PAIR, 1, n_pad), lambda i: (i, 0, 0)),
            pl.BlockSpec((_PAIR, 1, n_pad), lambda i: (i, 0, 0)),
            pl.BlockSpec((1, 1, 128), lambda i: (0, 0, 0)),
        ],
        out_specs=[
            pl.BlockSpec((_PAIR, 1, n_pad), lambda i: (i, 0, 0)),
            pl.BlockSpec((_PAIR, 1, 128), lambda i: (i, 0, 0)),
        ],
        out_shape=[
            jax.ShapeDtypeStruct((b, 1, n_pad), jnp.float32),
            jax.ShapeDtypeStruct((b, 1, 128), jnp.int32),
        ],
        scratch_shapes=[
            pltpu.VMEM((_PAIR, n_pad, n_pad), jnp.int32),
            pltpu.VMEM((_PAIR, n_pad, n_pad), jnp.bfloat16),
        ],
        compiler_params=pltpu.CompilerParams(
            dimension_semantics=("parallel",)),
    )(bx_p, by_p, bw_p, bh_p, prob_p, noise_p, scal)

    chosen = chosen_b.reshape(b, n_pad)[:, :n].T
    top_k_indices = idx_b.reshape(b, 128)[:, :_K].T
    batch_indices = jnp.broadcast_to(
        jnp.arange(b, dtype=top_k_indices.dtype).reshape(1, -1), (_K, b))
    return chosen, top_k_indices, batch_indices
